# Initial kernel scaffold; baseline (speedup 1.0000x reference)
import jax
import jax.numpy as jnp
from jax.experimental import pallas as pl


def kernel(uv, tex):
    raise NotImplementedError("write your pallas kernel here")


# speedup vs baseline: 23.7931x; 23.7931x over previous
import functools

import jax
import jax.numpy as jnp
from jax import lax
from jax.experimental import pallas as pl
from jax.experimental.pallas import tpu as pltpu
from jax.experimental.pallas import tpu_sc as plsc

H = W = 1024
M = 2097152
NC, NS, L = 2, 16, 16
NW = NC * NS
PW = M // NW
K = 1024
NCHUNK = PW // K
NV = K // L
GSUB = 128
NG = K // GSUB

_mesh = plsc.VectorSubcoreMesh(
    core_axis_name="c", subcore_axis_name="s", num_cores=NC, num_subcores=NS
)


@functools.partial(
    pl.kernel,
    out_type=[
        jax.ShapeDtypeStruct((3 * M,), jnp.float32),
        jax.ShapeDtypeStruct((M,), jnp.float32),
        jax.ShapeDtypeStruct((M,), jnp.float32),
    ],
    mesh=_mesh,
    scratch_types=[
        pltpu.VMEM((2 * K,), jnp.float32),
        pltpu.VMEM((K,), jnp.int32),
        pltpu.VMEM((K,), jnp.float32),
        pltpu.VMEM((K,), jnp.float32),
        pltpu.VMEM((K,), jnp.float32),
        pltpu.VMEM((K,), jnp.float32),
        pltpu.VMEM((K, 32), jnp.float32),
        pltpu.VMEM((3 * K,), jnp.float32),
        pltpu.VMEM((K,), jnp.float32),
        pltpu.VMEM((K,), jnp.float32),
        pltpu.SemaphoreType.DMA,
    ],
    compiler_params=pltpu.CompilerParams(
        needs_layout_passes=False, use_tc_tiling_on_sc=False),
)
def _sample_kernel(quad_hbm, uv_hbm, ab_hbm, met_hbm, rgh_hbm,
                   uv_v, key_v, w00_v, w01_v, w10_v, w11_v,
                   rows_v, ab_v, met_v, rgh_v, sem):
    wid = lax.axis_index("s") * NC + lax.axis_index("c")
    lanes = lax.iota(jnp.int32, L)

    def chunk_body(ci, carry):
        base = wid * PW + ci * K

        pltpu.sync_copy(uv_hbm.at[pl.ds(2 * base, 2 * K)], uv_v)

        def p1(i, c1):
            idx = lanes * 2 + (2 * L) * i
            ux = plsc.load_gather(uv_v, [idx])
            uy = plsc.load_gather(uv_v, [idx + 1])
            ix = ((ux * 2.0) * (0.5 * W)) - 0.5
            iy = ((uy * 2.0) * (0.5 * H)) - 0.5
            x0 = (ix + 1.0).astype(jnp.int32) - 1
            y0 = (iy + 1.0).astype(jnp.int32) - 1
            wx1 = ix - x0.astype(jnp.float32)
            wx0 = 1.0 - wx1
            wy1 = iy - y0.astype(jnp.float32)
            wy0 = 1.0 - wy1
            zero = jnp.zeros_like(ix)
            sx_lo = x0 < 0
            sx_hi = x0 > (W - 2)
            qx0 = jnp.where(sx_lo, wx1, jnp.where(sx_hi, zero, wx0))
            qx1 = jnp.where(sx_lo, zero, jnp.where(sx_hi, wx0, wx1))
            sy_lo = y0 < 0
            sy_hi = y0 > (H - 2)
            qy0 = jnp.where(sy_lo, wy1, jnp.where(sy_hi, zero, wy0))
            qy1 = jnp.where(sy_lo, zero, jnp.where(sy_hi, wy0, wy1))
            xk = jnp.clip(x0, 0, W - 2)
            yk = jnp.clip(y0, 0, H - 2)
            sl = pl.ds(i * L, L)
            key_v[sl] = yk * W + xk
            w00_v[sl] = qy0 * qx0
            w01_v[sl] = qy0 * qx1
            w10_v[sl] = qy1 * qx0
            w11_v[sl] = qy1 * qx1
            return c1

        lax.fori_loop(0, NV, p1, 0)

        copies = []
        for j in range(NG):
            copies.append(pltpu.async_copy(
                quad_hbm.at[key_v.at[pl.ds(j * GSUB, GSUB)]],
                rows_v.at[pl.ds(j * GSUB, GSUB)],
                sem,
            ))
        for cp in copies:
            cp.wait()

        def p2(i, c2):
            pbase = i * L
            prow = pbase + lanes
            sl = pl.ds(pbase, L)
            w00 = w00_v[sl]
            w01 = w01_v[sl]
            w10 = w10_v[sl]
            w11 = w11_v[sl]
            for c in range(5):
                col = jnp.full((L,), 4 * c, jnp.int32)
                v00 = plsc.load_gather(rows_v, [prow, col])
                v01 = plsc.load_gather(rows_v, [prow, col + 1])
                v10 = plsc.load_gather(rows_v, [prow, col + 2])
                v11 = plsc.load_gather(rows_v, [prow, col + 3])
                val = v00 * w00 + v01 * w01 + v10 * w10 + v11 * w11
                if c < 3:
                    plsc.store_scatter(
                        ab_v, [lanes * 3 + (3 * pbase + c)], val)
                elif c == 3:
                    met_v[sl] = val
                else:
                    rgh_v[sl] = val
            return c2

        lax.fori_loop(0, NV, p2, 0)

        pltpu.sync_copy(ab_v, ab_hbm.at[pl.ds(3 * base, 3 * K)])
        pltpu.sync_copy(met_v, met_hbm.at[pl.ds(base, K)])
        pltpu.sync_copy(rgh_v, rgh_hbm.at[pl.ds(base, K)])
        return carry

    lax.fori_loop(0, NCHUNK, chunk_body, 0)


def kernel(uv, tex):
    t = tex[0].transpose(1, 2, 0)
    t01 = jnp.roll(t, -1, axis=1)
    t10 = jnp.roll(t, -1, axis=0)
    t11 = jnp.roll(t10, -1, axis=1)
    quad = jnp.stack([t, t01, t10, t11], axis=-1).reshape(H * W, 20)
    quad = jnp.pad(quad, ((0, 0), (0, 12)))
    uvf = uv.reshape(-1)
    ab, met, rgh = _sample_kernel(quad, uvf)
    return (ab.reshape(M, 3), met.reshape(M, 1), rgh.reshape(M, 1))


# speedup vs baseline: 28.4435x; 1.1955x over previous
import functools

import jax
import jax.numpy as jnp
from jax import lax
from jax.experimental import pallas as pl
from jax.experimental.pallas import tpu as pltpu
from jax.experimental.pallas import tpu_sc as plsc

H = W = 1024
M = 2097152
NC, NS, L = 2, 16, 16
NW = NC * NS
PW = M // NW
K = 1024
NCHUNK = PW // K
NV = K // L
GSUB = 128
NG = K // GSUB

_mesh = plsc.VectorSubcoreMesh(
    core_axis_name="c", subcore_axis_name="s", num_cores=NC, num_subcores=NS
)

TEXROWS = H // NW


@functools.partial(
    pl.kernel,
    out_type=jax.ShapeDtypeStruct((H * W * 32,), jnp.float32),
    mesh=_mesh,
    scratch_types=[
        pltpu.VMEM((5, 2, W + 16), jnp.float32),
        pltpu.VMEM((W * 32,), jnp.float32),
    ],
    compiler_params=pltpu.CompilerParams(
        needs_layout_passes=False, use_tc_tiling_on_sc=False),
)
def _build_kernel(tex_hbm, quad_hbm, rows_in, out_v):
    wid = lax.axis_index("s") * NC + lax.axis_index("c")
    lanes = lax.iota(jnp.int32, L)

    def y_body(yi, carry):
        y = wid * TEXROWS + yi
        start = jnp.minimum(y, H - 2)
        pltpu.sync_copy(tex_hbm.at[:, pl.ds(start, 2), :],
                        rows_in.at[:, :, pl.ds(0, W)])

        def v_body(v, c2):
            xb = v * L
            base_idx = (xb + lanes) * 32
            for c in range(5):
                for j in range(2):
                    for i in range(2):
                        val = rows_in[c, j, pl.ds(xb + i, L)]
                        plsc.store_scatter(
                            out_v, [base_idx + (4 * c + 2 * j + i)], val)
            return c2

        lax.fori_loop(0, W // L, v_body, 0)
        pltpu.sync_copy(out_v, quad_hbm.at[pl.ds(y * (W * 32), W * 32)])
        return carry

    lax.fori_loop(0, TEXROWS, y_body, 0)


@functools.partial(
    pl.kernel,
    out_type=[
        jax.ShapeDtypeStruct((3 * M,), jnp.float32),
        jax.ShapeDtypeStruct((M,), jnp.float32),
        jax.ShapeDtypeStruct((M,), jnp.float32),
    ],
    mesh=_mesh,
    scratch_types=[
        pltpu.VMEM((2 * K,), jnp.float32),
        pltpu.VMEM((K,), jnp.int32),
        pltpu.VMEM((K,), jnp.float32),
        pltpu.VMEM((K,), jnp.float32),
        pltpu.VMEM((K,), jnp.float32),
        pltpu.VMEM((K,), jnp.float32),
        pltpu.VMEM((K, 32), jnp.float32),
        pltpu.VMEM((3 * K,), jnp.float32),
        pltpu.VMEM((K,), jnp.float32),
        pltpu.VMEM((K,), jnp.float32),
        pltpu.SemaphoreType.DMA,
    ],
    compiler_params=pltpu.CompilerParams(
        needs_layout_passes=False, use_tc_tiling_on_sc=False),
)
def _sample_kernel(quad_hbm, uv_hbm, ab_hbm, met_hbm, rgh_hbm,
                   uv_v, key_v, w00_v, w01_v, w10_v, w11_v,
                   rows_v, ab_v, met_v, rgh_v, sem):
    wid = lax.axis_index("s") * NC + lax.axis_index("c")
    lanes = lax.iota(jnp.int32, L)

    def chunk_body(ci, carry):
        base = wid * PW + ci * K

        pltpu.sync_copy(uv_hbm.at[pl.ds(2 * base, 2 * K)], uv_v)

        def p1(i, c1):
            idx = lanes * 2 + (2 * L) * i
            ux = plsc.load_gather(uv_v, [idx])
            uy = plsc.load_gather(uv_v, [idx + 1])
            ix = ((ux * 2.0) * (0.5 * W)) - 0.5
            iy = ((uy * 2.0) * (0.5 * H)) - 0.5
            x0 = (ix + 1.0).astype(jnp.int32) - 1
            y0 = (iy + 1.0).astype(jnp.int32) - 1
            wx1 = ix - x0.astype(jnp.float32)
            wx0 = 1.0 - wx1
            wy1 = iy - y0.astype(jnp.float32)
            wy0 = 1.0 - wy1
            zero = jnp.zeros_like(ix)
            sx_lo = x0 < 0
            sx_hi = x0 > (W - 2)
            qx0 = jnp.where(sx_lo, wx1, jnp.where(sx_hi, zero, wx0))
            qx1 = jnp.where(sx_lo, zero, jnp.where(sx_hi, wx0, wx1))
            sy_lo = y0 < 0
            sy_hi = y0 > (H - 2)
            qy0 = jnp.where(sy_lo, wy1, jnp.where(sy_hi, zero, wy0))
            qy1 = jnp.where(sy_lo, zero, jnp.where(sy_hi, wy0, wy1))
            xk = jnp.clip(x0, 0, W - 2)
            yk = jnp.clip(y0, 0, H - 2)
            sl = pl.ds(i * L, L)
            key_v[sl] = yk * W + xk
            w00_v[sl] = qy0 * qx0
            w01_v[sl] = qy0 * qx1
            w10_v[sl] = qy1 * qx0
            w11_v[sl] = qy1 * qx1
            return c1

        lax.fori_loop(0, NV, p1, 0)

        copies = []
        for j in range(NG):
            copies.append(pltpu.async_copy(
                quad_hbm.at[key_v.at[pl.ds(j * GSUB, GSUB)]],
                rows_v.at[pl.ds(j * GSUB, GSUB)],
                sem,
            ))
        for cp in copies:
            cp.wait()

        def p2(i, c2):
            pbase = i * L
            prow = pbase + lanes
            sl = pl.ds(pbase, L)
            w00 = w00_v[sl]
            w01 = w01_v[sl]
            w10 = w10_v[sl]
            w11 = w11_v[sl]
            for c in range(5):
                col = jnp.full((L,), 4 * c, jnp.int32)
                v00 = plsc.load_gather(rows_v, [prow, col])
                v01 = plsc.load_gather(rows_v, [prow, col + 1])
                v10 = plsc.load_gather(rows_v, [prow, col + 2])
                v11 = plsc.load_gather(rows_v, [prow, col + 3])
                val = v00 * w00 + v01 * w01 + v10 * w10 + v11 * w11
                if c < 3:
                    plsc.store_scatter(
                        ab_v, [lanes * 3 + (3 * pbase + c)], val)
                elif c == 3:
                    met_v[sl] = val
                else:
                    rgh_v[sl] = val
            return c2

        lax.fori_loop(0, NV, p2, 0)

        pltpu.sync_copy(ab_v, ab_hbm.at[pl.ds(3 * base, 3 * K)])
        pltpu.sync_copy(met_v, met_hbm.at[pl.ds(base, K)])
        pltpu.sync_copy(rgh_v, rgh_hbm.at[pl.ds(base, K)])
        return carry

    lax.fori_loop(0, NCHUNK, chunk_body, 0)


def kernel(uv, tex):
    quad = _build_kernel(tex[0]).reshape(H * W, 32)
    uvf = uv.reshape(-1)
    ab, met, rgh = _sample_kernel(quad, uvf)
    return (ab.reshape(M, 3), met.reshape(M, 1), rgh.reshape(M, 1))
